# Initial kernel scaffold; baseline (speedup 1.0000x reference)
import jax
import jax.numpy as jnp
from jax.experimental import pallas as pl


def kernel(x, edge_index, edge_weight, batch, W1, b1, W2, b2, Wc1, bc1, Wc2, bc2):
    raise NotImplementedError("write your pallas kernel here")


# speedup vs baseline: 9.8431x; 9.8431x over previous
import functools

import jax
import jax.numpy as jnp
from jax import lax
from jax.experimental import pallas as pl
from jax.experimental.pallas import tpu as pltpu
from jax.experimental.pallas import tpu_sc as plsc

N = 10000
NP = 10240
E = 320000
D = 128
H = 64
G = 64
NC = 2
NS = 16
NW = NC * NS
E_PER = E // NW
CHUNK = 80
NCHUNK = E_PER // CHUNK
DEG_BLK = 2000
ROWS16 = NP // 16
RPT = ROWS16 // NS

_sc_mesh = plsc.VectorSubcoreMesh(core_axis_name="c", subcore_axis_name="s")

_Z16 = None


def _iota16(off):
    return lax.broadcasted_iota(jnp.int32, (16,), 0) + off


@functools.partial(
    pl.kernel,
    out_type=jax.ShapeDtypeStruct((NW, NP), jnp.float32),
    mesh=_sc_mesh,
    scratch_types=[
        pltpu.VMEM((NP,), jnp.float32),
        pltpu.VMEM((DEG_BLK,), jnp.int32),
        pltpu.VMEM((DEG_BLK,), jnp.float32),
    ],
    compiler_params=pltpu.CompilerParams(needs_layout_passes=False),
)
def _sc_deg(dst_hbm, ew_hbm, out_hbm, degp, dstb, ewb):
    cid = lax.axis_index("c")
    sid = lax.axis_index("s")
    wid = sid * NC + cid
    z16 = jnp.zeros((16,), jnp.float32)

    def zero_body(i, _):
        degp[pl.ds(i * 16, 16)] = z16
        return 0
    lax.fori_loop(0, NP // 16, zero_body, 0)

    def blk_body(c, _):
        base = wid * E_PER + c * DEG_BLK
        pltpu.sync_copy(dst_hbm.at[pl.ds(base, DEG_BLK)], dstb)
        pltpu.sync_copy(ew_hbm.at[pl.ds(base, DEG_BLK)], ewb)

        def vec_body(j, _):
            d = dstb[pl.ds(j * 16, 16)]
            w = ewb[pl.ds(j * 16, 16)]
            plsc.addupdate_scatter(degp, [d], w)
            return 0
        lax.fori_loop(0, DEG_BLK // 16, vec_body, 0)
        return 0
    lax.fori_loop(0, E_PER // DEG_BLK, blk_body, 0)

    pltpu.sync_copy(degp, out_hbm.at[wid])


@functools.partial(
    pl.kernel,
    out_type=jax.ShapeDtypeStruct((NC, NP, H), jnp.float32),
    mesh=_sc_mesh,
    scratch_types=[
        pltpu.VMEM((CHUNK,), jnp.int32),
        pltpu.VMEM((CHUNK,), jnp.int32),
        pltpu.VMEM((CHUNK,), jnp.float32),
        pltpu.VMEM((CHUNK, H), jnp.float32),
        pltpu.VMEM((128, H), jnp.float32),
        pltpu.VMEM_SHARED((NP, H), jnp.float32),
        pltpu.SemaphoreType.DMA,
    ],
    compiler_params=pltpu.CompilerParams(needs_layout_passes=False,
                                         use_tc_tiling_on_sc=False),
)
def _sc_agg(g_hbm, src_hbm, dst_hbm, ew_hbm, out_hbm,
            srcb, dstb, ewb, rows, zb, acc_sh, sem):
    cid = lax.axis_index("c")
    sid = lax.axis_index("s")
    wid = sid * NC + cid
    z16 = jnp.zeros((16,), jnp.float32)

    def zzb(i, _):
        r = i >> 2
        q = i & 3
        zb[r, pl.ds(q * 16, 16)] = z16
        return 0
    lax.fori_loop(0, 128 * (H // 16), zzb, 0)
    for c in range(NP // NS // 128):
        pltpu.sync_copy(zb, acc_sh.at[pl.ds(sid * (NP // NS) + c * 128, 128)])

    plsc.subcore_barrier()

    def chunk_body(c, _):
        base = wid * E_PER + c * CHUNK
        pltpu.sync_copy(src_hbm.at[pl.ds(base, CHUNK)], srcb)
        pltpu.sync_copy(dst_hbm.at[pl.ds(base, CHUNK)], dstb)
        pltpu.sync_copy(ew_hbm.at[pl.ds(base, CHUNK)], ewb)
        pltpu.async_copy(g_hbm.at[srcb], rows, sem).wait()

        def scale_body(kb, _):
            wv = ewb[pl.ds(kb * 16, 16)]
            for lane in range(16):
                k = kb * 16 + lane
                w = wv[lane]
                for j in range(H // 16):
                    v = rows[k, pl.ds(j * 16, 16)]
                    rows[k, pl.ds(j * 16, 16)] = v * w
            return 0
        lax.fori_loop(0, CHUNK // 16, scale_body, 0)

        pltpu.sync_copy(rows, acc_sh.at[dstb], add=True)
        return 0
    lax.fori_loop(0, NCHUNK, chunk_body, 0)

    plsc.subcore_barrier()

    pltpu.sync_copy(acc_sh.at[pl.ds(sid * (NP // NS), NP // NS)],
                    out_hbm.at[cid, pl.ds(sid * (NP // NS), NP // NS)])


def _tc_a_body(x_ref, w1_ref, deg32_ref, g_ref, dinv_ref):
    ones = jnp.ones((NW, 1), jnp.float32)
    deg = lax.dot_general(deg32_ref[...], ones, (((0,), (0,)), ((), ())),
                          preferred_element_type=jnp.float32)
    dinv = lax.rsqrt(deg + 1.0)
    dinv_ref[...] = dinv
    h = jnp.dot(x_ref[...], w1_ref[...], preferred_element_type=jnp.float32)
    g_ref[...] = h * dinv


_tc_a = pl.pallas_call(
    _tc_a_body,
    out_shape=[jax.ShapeDtypeStruct((NP, H), jnp.float32),
               jax.ShapeDtypeStruct((NP, 1), jnp.float32)],
)


def _tc_b_body(a0_ref, a1_ref, g1_ref, dinv_ref, b1_ref, w2_ref, g2_ref):
    dinv = dinv_ref[...]
    o = (a0_ref[...] + a1_ref[...] + g1_ref[...]) * dinv + b1_ref[...]
    o = jnp.maximum(o, 0.0)
    h2 = jnp.dot(o, w2_ref[...], preferred_element_type=jnp.float32)
    g2_ref[...] = h2 * dinv


_tc_b = pl.pallas_call(
    _tc_b_body,
    out_shape=jax.ShapeDtypeStruct((NP, H), jnp.float32),
)


def _tc_c_body(a0_ref, a1_ref, g2_ref, dinv_ref, b2_ref, batch_ref,
               wc1_ref, bc1_ref, wc2_ref, bc2_ref, out_ref):
    o = (a0_ref[...] + a1_ref[...] + g2_ref[...]) * dinv_ref[...] + b2_ref[...]
    o = jnp.maximum(o, 0.0)
    b = batch_ref[...]
    gid = lax.broadcasted_iota(jnp.int32, (G, NP), 0)
    p = (b == gid).astype(jnp.float32)
    s = jnp.dot(p, o, preferred_element_type=jnp.float32)
    cnt = jnp.sum(p, axis=1, keepdims=True)
    mean = s / jnp.maximum(cnt, 1.0)
    z = jnp.dot(mean, wc1_ref[...], preferred_element_type=jnp.float32)
    z = jnp.maximum(z + bc1_ref[...], 0.0)
    out_ref[...] = (jnp.dot(z, wc2_ref[...],
                            preferred_element_type=jnp.float32) + bc2_ref[...])


_tc_c = pl.pallas_call(
    _tc_c_body,
    out_shape=jax.ShapeDtypeStruct((G, 128), jnp.float32),
)


def kernel(x, edge_index, edge_weight, batch, W1, b1, W2, b2, Wc1, bc1, Wc2, bc2):
    src = edge_index[0]
    dst = edge_index[1]
    xp = jnp.pad(x, ((0, NP - N), (0, 0)))
    batch_p = jnp.pad(batch, (0, NP - N), constant_values=-1).reshape(1, NP)

    deg32 = _sc_deg(dst, edge_weight)
    g1, dinv = _tc_a(xp, W1, deg32)
    acc1 = _sc_agg(g1, src, dst, edge_weight)
    g2 = _tc_b(acc1[0], acc1[1], g1, dinv, b1.reshape(1, H), W2)
    acc2 = _sc_agg(g2, src, dst, edge_weight)

    wc1p = jnp.pad(Wc1, ((0, 0), (0, 128 - H // 2)))
    bc1p = jnp.pad(bc1, (0, 128 - H // 2)).reshape(1, 128)
    wc2p = jnp.pad(Wc2, ((0, 128 - H // 2), (0, 126)))
    bc2p = jnp.pad(bc2, (0, 126)).reshape(1, 128)
    outp = _tc_c(acc2[0], acc2[1], g2, dinv, b2.reshape(1, H), batch_p,
                 wc1p, bc1p, wc2p, bc2p)
    return outp[:, :2]
